# baseline (speedup 1.0000x reference)
import functools

import jax
import jax.numpy as jnp
from jax import lax
from jax.experimental import pallas as pl
from jax.experimental.pallas import tpu as pltpu
from jax.experimental.pallas import tpu_sc as plsc

NC = 2
NS = 16
LANES = 16

IDX_W = 128


@functools.lru_cache(maxsize=None)
def _build(B, L, V):
    CH = 16
    ROWS_T = B // NS
    NCH = ROWS_T // CH
    N = CH * L
    NJ = N // IDX_W
    assert B % NS == 0 and ROWS_T % CH == 0 and N % IDX_W == 0

    inv_l = 1.0 / float(L)

    mesh = plsc.VectorSubcoreMesh(core_axis_name="c", subcore_axis_name="s")
    vm = pltpu.VMEM

    @functools.partial(
        pl.kernel,
        out_type=jax.ShapeDtypeStruct((2 * B,), jnp.float32),
        mesh=mesh,
        scratch_types=[
            vm((N,), jnp.int32), vm((N,), jnp.int32),
            vm((N,), jnp.int32), vm((N,), jnp.int32),
            vm((N,), jnp.int32), vm((N,), jnp.int32),
            vm((N,), jnp.float32), vm((N,), jnp.float32),
            vm((N,), jnp.float32), vm((N,), jnp.float32),
            vm((N,), jnp.float32), vm((N,), jnp.float32),
            vm((LANES,), jnp.float32),
            pltpu.SemaphoreType.DMA, pltpu.SemaphoreType.DMA,
        ],
        compiler_params=pltpu.CompilerParams(needs_layout_passes=False),
    )
    def sc_kernel(xi_hbm, yi_hbm, xv_hbm, yv_hbm, wpair, upair,
                  out_hbm,
                  ix0, ix1, ia0, ia1, ib0, ib1, vv0, vv1,
                  gtf0, gtf1, gti0, gti1, outv, sem0, sem1):
        ixs, ias, ibs = (ix0, ix1), (ia0, ia1), (ib0, ib1)
        vvs, gtfs, gtis = (vv0, vv1), (gtf0, gtf1), (gti0, gti1)
        sems = (sem0, sem1)

        cid = lax.axis_index("c")
        sid = lax.axis_index("s")
        row_lanes = lax.iota(jnp.int32, LANES) * L

        def pipeline(idx_hbm, val_hbm, pairs, out_base):
            tbase = sid * ROWS_T * L

            def stage(c):
                p = c % 2
                base = tbase + c * N
                pltpu.sync_copy(idx_hbm.at[pl.ds(base, N)], ixs[p])
                pltpu.sync_copy(val_hbm.at[pl.ds(base, N)], vvs[p])

                def tbody(r, carry):
                    sl = pl.ds(r * LANES, LANES)
                    v2 = ixs[p][sl] * 2
                    ias[p][sl] = v2
                    ibs[p][sl] = v2 + 1
                    return carry
                lax.fori_loop(0, N // LANES, tbody, 0, unroll=8)

            def fire(c):
                p = c % 2

                def body(j, carry):
                    win = pl.ds(j * IDX_W, IDX_W)
                    pltpu.async_copy(pairs.at[ias[p].at[win]], gtfs[p].at[win], sems[p])
                    pltpu.async_copy(pairs.at[ibs[p].at[win]], gtis[p].at[win], sems[p])
                    return carry
                lax.fori_loop(0, NJ, body, 0)

            def drain(c):
                p = c % 2
                pltpu.make_async_copy(pairs.at[pl.ds(0, N)], gtfs[p], sems[p]).wait()
                pltpu.make_async_copy(pairs.at[pl.ds(0, N)], gtis[p], sems[p]).wait()

            def compute(c):
                p = c % 2

                def cbody(j, carry):
                    a_tf, a_ti = carry
                    idxv = row_lanes + j
                    vtf = plsc.load_gather(gtfs[p], [idxv])
                    vti = plsc.load_gather(gtis[p], [idxv])
                    vq = plsc.load_gather(vvs[p], [idxv])
                    return (a_tf + vtf, a_ti + vti * vq)
                zero = jnp.zeros((LANES,), jnp.float32)
                a_tf, a_ti = lax.fori_loop(0, L, cbody, (zero, zero), unroll=4)

                outv[...] = a_tf * inv_l + a_ti
                dst = pl.ds(out_base + sid * ROWS_T + c * CH, CH)
                pltpu.sync_copy(outv, out_hbm.at[dst])

            stage(0)
            fire(0)
            for c in range(NCH):
                if c + 1 < NCH:
                    stage(c + 1)
                    fire(c + 1)
                drain(c)
                compute(c)

        @pl.when(cid == 0)
        def _():
            pipeline(xi_hbm, xv_hbm, wpair, 0)

        @pl.when(cid == 1)
        def _():
            pipeline(yi_hbm, yv_hbm, upair, B)

    bias = 0.0 + 0.0001 * float(L) / 10.0 + 0.0001 * float(L) / 10.0

    def fin_body(p_ref, o_ref):
        z = p_ref[0, :] + p_ref[1, :] + bias
        o_ref[...] = 1.0 / (1.0 + jnp.exp(-z))

    finisher = pl.pallas_call(
        fin_body,
        out_shape=jax.ShapeDtypeStruct((B,), jnp.float32),
    )

    def run(xi, yi, xvf, yvf, wpair, upair):
        partials = sc_kernel(xi, yi, xvf, yvf, wpair, upair)
        return finisher(partials.reshape(2, B))

    return run


def kernel(x_idx, y_idx, x_tfidf, y_tfidf, W_tf, U_tf, W_tfidf, U_tfidf):
    B, L = x_idx.shape
    V = W_tf.shape[0]
    wpair = jnp.stack([W_tf, W_tfidf], axis=1).reshape(-1)
    upair = jnp.stack([U_tf, U_tfidf], axis=1).reshape(-1)
    xi = x_idx.reshape(-1)
    yi = y_idx.reshape(-1)
    xvf = x_tfidf.reshape(-1)
    yvf = y_tfidf.reshape(-1)
    return _build(B, L, V)(xi, yi, xvf, yvf, wpair, upair)


# speedup vs baseline: 1.0002x; 1.0002x over previous
import functools

import jax
import jax.numpy as jnp
from jax import lax
from jax.experimental import pallas as pl
from jax.experimental.pallas import tpu as pltpu
from jax.experimental.pallas import tpu_sc as plsc

NC = 2
NS = 16
LANES = 16

IDX_W = 128


@functools.lru_cache(maxsize=None)
def _build(B, L, V):
    CH = 16
    ROWS_T = B // NS
    NCH = ROWS_T // CH
    N = CH * L
    NJ = N // IDX_W
    assert B % NS == 0 and ROWS_T % CH == 0 and N % IDX_W == 0

    inv_l = 1.0 / float(L)

    mesh = plsc.VectorSubcoreMesh(core_axis_name="c", subcore_axis_name="s")
    vm = pltpu.VMEM

    @functools.partial(
        pl.kernel,
        out_type=jax.ShapeDtypeStruct((2 * B,), jnp.float32),
        mesh=mesh,
        scratch_types=[
            vm((N,), jnp.int32), vm((N,), jnp.int32),
            vm((2 * N,), jnp.int32), vm((2 * N,), jnp.int32),
            vm((N,), jnp.float32), vm((N,), jnp.float32),
            vm((2 * N,), jnp.float32), vm((2 * N,), jnp.float32),
            vm((LANES,), jnp.float32),
            pltpu.SemaphoreType.DMA, pltpu.SemaphoreType.DMA,
        ],
        compiler_params=pltpu.CompilerParams(needs_layout_passes=False),
    )
    def sc_kernel(xi_hbm, yi_hbm, xv_hbm, yv_hbm, wpair, upair,
                  out_hbm,
                  ix0, ix1, ip0, ip1, vv0, vv1,
                  gp0, gp1, outv, sem0, sem1):
        ixs, iprs = (ix0, ix1), (ip0, ip1)
        vvs, gprs = (vv0, vv1), (gp0, gp1)
        sems = (sem0, sem1)

        cid = lax.axis_index("c")
        sid = lax.axis_index("s")
        row_lanes = lax.iota(jnp.int32, LANES) * L
        lane2 = lax.iota(jnp.int32, LANES) * 2

        def pipeline(idx_hbm, val_hbm, pairs, out_base):
            tbase = sid * ROWS_T * L

            def stage(c):
                p = c % 2
                base = tbase + c * N
                pltpu.sync_copy(idx_hbm.at[pl.ds(base, N)], ixs[p])
                pltpu.sync_copy(val_hbm.at[pl.ds(base, N)], vvs[p])

                def tbody(r, carry):
                    sl = pl.ds(r * LANES, LANES)
                    v2 = ixs[p][sl] * 2
                    rows = lane2 + r * (2 * LANES)
                    plsc.store_scatter(iprs[p], [rows], v2)
                    plsc.store_scatter(iprs[p], [rows + 1], v2 + 1)
                    return carry
                lax.fori_loop(0, N // LANES, tbody, 0, unroll=8)

            def fire(c):
                p = c % 2

                def body(j, carry):
                    win = pl.ds(j * IDX_W, IDX_W)
                    pltpu.async_copy(pairs.at[iprs[p].at[win]], gprs[p].at[win], sems[p])
                    return carry
                lax.fori_loop(0, 2 * NJ, body, 0)

            def drain(c):
                p = c % 2
                pltpu.make_async_copy(pairs.at[pl.ds(0, 2 * N)], gprs[p], sems[p]).wait()

            def compute(c):
                p = c % 2

                def cbody(j, carry):
                    a_tf, a_ti = carry
                    idxv = row_lanes + j
                    idx2 = idxv * 2
                    vtf = plsc.load_gather(gprs[p], [idx2])
                    vti = plsc.load_gather(gprs[p], [idx2 + 1])
                    vq = plsc.load_gather(vvs[p], [idxv])
                    return (a_tf + vtf, a_ti + vti * vq)
                zero = jnp.zeros((LANES,), jnp.float32)
                a_tf, a_ti = lax.fori_loop(0, L, cbody, (zero, zero), unroll=4)

                outv[...] = a_tf * inv_l + a_ti
                dst = pl.ds(out_base + sid * ROWS_T + c * CH, CH)
                pltpu.sync_copy(outv, out_hbm.at[dst])

            stage(0)
            fire(0)
            for c in range(NCH):
                if c + 1 < NCH:
                    stage(c + 1)
                    fire(c + 1)
                drain(c)
                compute(c)

        @pl.when(cid == 0)
        def _():
            pipeline(xi_hbm, xv_hbm, wpair, 0)

        @pl.when(cid == 1)
        def _():
            pipeline(yi_hbm, yv_hbm, upair, B)

    bias = 0.0 + 0.0001 * float(L) / 10.0 + 0.0001 * float(L) / 10.0

    def fin_body(p_ref, o_ref):
        z = p_ref[0, :] + p_ref[1, :] + bias
        o_ref[...] = 1.0 / (1.0 + jnp.exp(-z))

    finisher = pl.pallas_call(
        fin_body,
        out_shape=jax.ShapeDtypeStruct((B,), jnp.float32),
    )

    def run(xi, yi, xvf, yvf, wpair, upair):
        partials = sc_kernel(xi, yi, xvf, yvf, wpair, upair)
        return finisher(partials.reshape(2, B))

    return run


def kernel(x_idx, y_idx, x_tfidf, y_tfidf, W_tf, U_tf, W_tfidf, U_tfidf):
    B, L = x_idx.shape
    V = W_tf.shape[0]
    wpair = jnp.stack([W_tf, W_tfidf], axis=1).reshape(-1)
    upair = jnp.stack([U_tf, U_tfidf], axis=1).reshape(-1)
    xi = x_idx.reshape(-1)
    yi = y_idx.reshape(-1)
    xvf = x_tfidf.reshape(-1)
    yvf = y_tfidf.reshape(-1)
    return _build(B, L, V)(xi, yi, xvf, yvf, wpair, upair)


# speedup vs baseline: 10.4753x; 10.4737x over previous
import functools

import jax
import jax.numpy as jnp
from jax import lax
from jax.experimental import pallas as pl
from jax.experimental.pallas import tpu as pltpu
from jax.experimental.pallas import tpu_sc as plsc

NC = 2
NS = 16
LANES = 16

IDX_W = 128


@functools.lru_cache(maxsize=None)
def _build(B, L, V):
    CH = 16
    ROWS_T = B // NS
    NCH = ROWS_T // CH
    N = CH * L
    NJ = N // IDX_W
    assert B % NS == 0 and ROWS_T % CH == 0 and N % IDX_W == 0 and V % 8 == 0

    VCH = (V // (8 * NS)) * 8
    VTAIL = V - NS * VCH
    NIB = 6
    KB = VCH // NIB
    assert VCH % NIB == 0 and KB % LANES == 0 and KB % 8 == 0
    assert VTAIL % LANES == 0 and VTAIL <= KB

    inv_l = 1.0 / float(L)

    mesh = plsc.VectorSubcoreMesh(core_axis_name="c", subcore_axis_name="s")
    vm = pltpu.VMEM

    @functools.partial(
        pl.kernel,
        out_type=(
            jax.ShapeDtypeStruct((2 * B,), jnp.float32),
            jax.ShapeDtypeStruct((2 * V,), jnp.float32),
            jax.ShapeDtypeStruct((2 * V,), jnp.float32),
        ),
        mesh=mesh,
        scratch_types=[
            vm((KB,), jnp.float32), vm((KB,), jnp.float32),
            vm((2 * KB,), jnp.float32),
            vm((N,), jnp.int32), vm((N,), jnp.int32),
            vm((2 * N,), jnp.int32), vm((2 * N,), jnp.int32),
            vm((N,), jnp.float32), vm((N,), jnp.float32),
            vm((2 * N,), jnp.float32), vm((2 * N,), jnp.float32),
            vm((LANES,), jnp.float32),
            pltpu.SemaphoreType.DMA, pltpu.SemaphoreType.DMA,
        ],
        compiler_params=pltpu.CompilerParams(needs_layout_passes=False),
    )
    def sc_kernel(xi_hbm, yi_hbm, xv_hbm, yv_hbm, wtf, wti, utf, uti,
                  out_hbm, wpair, upair,
                  tfb, tib, pairb,
                  ix0, ix1, ip0, ip1, vv0, vv1,
                  gp0, gp1, outv, sem0, sem1):
        ixs, iprs = (ix0, ix1), (ip0, ip1)
        vvs, gprs = (vv0, vv1), (gp0, gp1)
        sems = (sem0, sem1)

        cid = lax.axis_index("c")
        sid = lax.axis_index("s")
        row_lanes = lax.iota(jnp.int32, LANES) * L
        lane2 = lax.iota(jnp.int32, LANES) * 2

        def interleave_block(t_tf, t_ti, pairs, off, size):
            pltpu.sync_copy(t_tf.at[pl.ds(off, size)], tfb.at[pl.ds(0, size)])
            pltpu.sync_copy(t_ti.at[pl.ds(off, size)], tib.at[pl.ds(0, size)])

            def body(r, carry):
                sl = pl.ds(r * LANES, LANES)
                rows = lane2 + r * (2 * LANES)
                plsc.store_scatter(pairb, [rows], tfb[sl])
                plsc.store_scatter(pairb, [rows + 1], tib[sl])
                return carry
            lax.fori_loop(0, size // LANES, body, 0, unroll=8)
            pltpu.sync_copy(pairb.at[pl.ds(0, 2 * size)],
                            pairs.at[pl.ds(2 * off, 2 * size)])

        def interleave(t_tf, t_ti, pairs):
            for k in range(NIB):
                interleave_block(t_tf, t_ti, pairs, sid * VCH + k * KB, KB)
            if VTAIL:
                @pl.when(sid == NS - 1)
                def _():
                    interleave_block(t_tf, t_ti, pairs, NS * VCH, VTAIL)

        def pipeline(idx_hbm, val_hbm, pairs, out_base):
            tbase = sid * ROWS_T * L

            def stage(c):
                p = c % 2
                base = tbase + c * N
                pltpu.sync_copy(idx_hbm.at[pl.ds(base, N)], ixs[p])
                pltpu.sync_copy(val_hbm.at[pl.ds(base, N)], vvs[p])

                def tbody(r, carry):
                    sl = pl.ds(r * LANES, LANES)
                    v2 = ixs[p][sl] * 2
                    rows = lane2 + r * (2 * LANES)
                    plsc.store_scatter(iprs[p], [rows], v2)
                    plsc.store_scatter(iprs[p], [rows + 1], v2 + 1)
                    return carry
                lax.fori_loop(0, N // LANES, tbody, 0, unroll=8)

            def fire(c):
                p = c % 2

                def body(j, carry):
                    win = pl.ds(j * IDX_W, IDX_W)
                    pltpu.async_copy(pairs.at[iprs[p].at[win]], gprs[p].at[win], sems[p])
                    return carry
                lax.fori_loop(0, 2 * NJ, body, 0)

            def drain(c):
                p = c % 2
                pltpu.make_async_copy(pairs.at[pl.ds(0, 2 * N)], gprs[p], sems[p]).wait()

            def compute(c):
                p = c % 2

                def cbody(j, carry):
                    a_tf, a_ti = carry
                    idxv = row_lanes + j
                    idx2 = idxv * 2
                    vtf = plsc.load_gather(gprs[p], [idx2])
                    vti = plsc.load_gather(gprs[p], [idx2 + 1])
                    vq = plsc.load_gather(vvs[p], [idxv])
                    return (a_tf + vtf, a_ti + vti * vq)
                zero = jnp.zeros((LANES,), jnp.float32)
                a_tf, a_ti = lax.fori_loop(0, L, cbody, (zero, zero), unroll=4)

                outv[...] = a_tf * inv_l + a_ti
                dst = pl.ds(out_base + sid * ROWS_T + c * CH, CH)
                pltpu.sync_copy(outv, out_hbm.at[dst])

            stage(0)
            fire(0)
            for c in range(NCH):
                if c + 1 < NCH:
                    stage(c + 1)
                    fire(c + 1)
                drain(c)
                compute(c)

        @pl.when(cid == 0)
        def _():
            interleave(wtf, wti, wpair)

        @pl.when(cid == 1)
        def _():
            interleave(utf, uti, upair)

        plsc.subcore_barrier()

        @pl.when(cid == 0)
        def _():
            pipeline(xi_hbm, xv_hbm, wpair, 0)

        @pl.when(cid == 1)
        def _():
            pipeline(yi_hbm, yv_hbm, upair, B)

    bias = 0.0 + 0.0001 * float(L) / 10.0 + 0.0001 * float(L) / 10.0

    def fin_body(p_ref, o_ref):
        z = p_ref[0, :] + p_ref[1, :] + bias
        o_ref[...] = 1.0 / (1.0 + jnp.exp(-z))

    finisher = pl.pallas_call(
        fin_body,
        out_shape=jax.ShapeDtypeStruct((B,), jnp.float32),
    )

    def run(xi, yi, xvf, yvf, W_tf, U_tf, W_tfidf, U_tfidf):
        partials, _, _ = sc_kernel(xi, yi, xvf, yvf,
                                   W_tf, W_tfidf, U_tf, U_tfidf)
        return finisher(partials.reshape(2, B))

    return run


def kernel(x_idx, y_idx, x_tfidf, y_tfidf, W_tf, U_tf, W_tfidf, U_tfidf):
    B, L = x_idx.shape
    V = W_tf.shape[0]
    xi = x_idx.reshape(-1)
    yi = y_idx.reshape(-1)
    xvf = x_tfidf.reshape(-1)
    yvf = y_tfidf.reshape(-1)
    return _build(B, L, V)(xi, yi, xvf, yvf, W_tf, U_tf, W_tfidf, U_tfidf)


# speedup vs baseline: 15.2363x; 1.4545x over previous
import functools

import jax
import jax.numpy as jnp
from jax import lax
from jax.experimental import pallas as pl
from jax.experimental.pallas import tpu as pltpu
from jax.experimental.pallas import tpu_sc as plsc

NC = 2
NS = 16
LANES = 16

IDX_W = 128


@functools.lru_cache(maxsize=None)
def _build(B, L, V):
    CH = 16
    ROWS_T = B // NS
    NCH = ROWS_T // CH
    N = CH * L
    LTAIL = L - IDX_W
    assert B % NS == 0 and ROWS_T % CH == 0
    assert 0 < LTAIL <= IDX_W and LTAIL % 8 == 0 and IDX_W % 8 == 0

    inv_l = 1.0 / float(L)

    mesh = plsc.VectorSubcoreMesh(core_axis_name="c", subcore_axis_name="s")
    vm = pltpu.VMEM

    @functools.partial(
        pl.kernel,
        out_type=jax.ShapeDtypeStruct((2 * B,), jnp.float32),
        mesh=mesh,
        scratch_types=[
            vm((CH, L), jnp.int32), vm((CH, L), jnp.int32),
            vm((CH, L), jnp.float32), vm((CH, L), jnp.float32),
            vm((N,), jnp.float32), vm((N,), jnp.float32),
            vm((N,), jnp.float32), vm((N,), jnp.float32),
            vm((LANES,), jnp.float32),
            pltpu.SemaphoreType.DMA, pltpu.SemaphoreType.DMA,
        ],
        compiler_params=pltpu.CompilerParams(needs_layout_passes=False),
    )
    def sc_kernel(xi_hbm, yi_hbm, xv_hbm, yv_hbm, wtf, wti, utf, uti,
                  out_hbm,
                  ix0, ix1, vv0, vv1, gtf0, gtf1, gti0, gti1,
                  outv, sem0, sem1):
        ixs, vvs = (ix0, ix1), (vv0, vv1)
        gtfs, gtis, sems = (gtf0, gtf1), (gti0, gti1), (sem0, sem1)

        cid = lax.axis_index("c")
        sid = lax.axis_index("s")
        row_lanes = lax.iota(jnp.int32, LANES) * L
        lane_iota = lax.iota(jnp.int32, LANES)

        def pipeline(idx_hbm, val_hbm, t_tf, t_ti, out_base):
            def stage(c):
                p = c % 2
                r0 = sid * ROWS_T + c * CH
                pltpu.sync_copy(idx_hbm.at[pl.ds(r0, CH), :], ixs[p])
                pltpu.sync_copy(val_hbm.at[pl.ds(r0, CH), :], vvs[p])

            def fire(c):
                p = c % 2

                def body(r, carry):
                    iw0 = ixs[p].at[r, pl.ds(0, IDX_W)]
                    iw1 = ixs[p].at[r, pl.ds(IDX_W, LTAIL)]
                    d0 = pl.ds(r * L, IDX_W)
                    d1 = pl.ds(r * L + IDX_W, LTAIL)
                    pltpu.async_copy(t_tf.at[iw0], gtfs[p].at[d0], sems[p])
                    pltpu.async_copy(t_tf.at[iw1], gtfs[p].at[d1], sems[p])
                    pltpu.async_copy(t_ti.at[iw0], gtis[p].at[d0], sems[p])
                    pltpu.async_copy(t_ti.at[iw1], gtis[p].at[d1], sems[p])
                    return carry
                lax.fori_loop(0, CH, body, 0)

            def drain(c):
                p = c % 2
                pltpu.make_async_copy(t_tf.at[pl.ds(0, N)], gtfs[p], sems[p]).wait()
                pltpu.make_async_copy(t_ti.at[pl.ds(0, N)], gtis[p], sems[p]).wait()

            def compute(c):
                p = c % 2

                def cbody(j, carry):
                    a_tf, a_ti = carry
                    idxv = row_lanes + j
                    jcol = lane_iota * 0 + j
                    vtf = plsc.load_gather(gtfs[p], [idxv])
                    vti = plsc.load_gather(gtis[p], [idxv])
                    vq = plsc.load_gather(vvs[p], [lane_iota, jcol])
                    return (a_tf + vtf, a_ti + vti * vq)
                zero = jnp.zeros((LANES,), jnp.float32)
                a_tf, a_ti = lax.fori_loop(0, L, cbody, (zero, zero), unroll=4)

                outv[...] = a_tf * inv_l + a_ti
                dst = pl.ds(out_base + sid * ROWS_T + c * CH, CH)
                pltpu.sync_copy(outv, out_hbm.at[dst])

            stage(0)
            fire(0)
            for c in range(NCH):
                if c + 1 < NCH:
                    stage(c + 1)
                    fire(c + 1)
                drain(c)
                compute(c)

        @pl.when(cid == 0)
        def _():
            pipeline(xi_hbm, xv_hbm, wtf, wti, 0)

        @pl.when(cid == 1)
        def _():
            pipeline(yi_hbm, yv_hbm, utf, uti, B)

    bias = 0.0 + 0.0001 * float(L) / 10.0 + 0.0001 * float(L) / 10.0

    def fin_body(p_ref, o_ref):
        z = p_ref[0, :] + p_ref[1, :] + bias
        o_ref[...] = 1.0 / (1.0 + jnp.exp(-z))

    finisher = pl.pallas_call(
        fin_body,
        out_shape=jax.ShapeDtypeStruct((B,), jnp.float32),
    )

    def run(x_idx, y_idx, x_tfidf, y_tfidf, W_tf, U_tf, W_tfidf, U_tfidf):
        partials = sc_kernel(x_idx, y_idx, x_tfidf, y_tfidf,
                             W_tf, W_tfidf, U_tf, U_tfidf)
        return finisher(partials.reshape(2, B))

    return run


def kernel(x_idx, y_idx, x_tfidf, y_tfidf, W_tf, U_tf, W_tfidf, U_tfidf):
    B, L = x_idx.shape
    V = W_tf.shape[0]
    return _build(B, L, V)(x_idx, y_idx, x_tfidf, y_tfidf,
                           W_tf, U_tf, W_tfidf, U_tfidf)
